# baseline (speedup 1.0000x reference)
import dataclasses

import jax
import jax.numpy as jnp
from jax import lax
from jax.experimental import pallas as pl
from jax.experimental.pallas import tpu as pltpu
from jax.experimental.pallas import tpu_sc as plsc

N = 10000
E = 320000
B = 64
D = 128
NC = 10

NP_ = 10112
HD = 64
NSUB = 16
CHK = 164
CHK2 = CHK // 2
K = 128
EP = NSUB * CHK * K
EH = CHK2 * K
RPS = NP_ // 16
NB = 2
NGRPH = CHK2 // NB


def _tc_fwd_body(hin_ref, w_ref, asrc_ref, adst_ref, h_ref, sd_ref):
    h = jnp.dot(hin_ref[...], w_ref[...], preferred_element_type=jnp.float32)
    h_ref[0, :, :] = h[:, 0:HD]
    h_ref[1, :, :] = h[:, HD:D]
    sd_ref[0, :] = jnp.sum(h * asrc_ref[...], axis=1)
    sd_ref[1, :] = jnp.sum(h * adst_ref[...], axis=1)


def _tc_fwd(hin, w, asrc, adst):
    return pl.pallas_call(
        _tc_fwd_body,
        out_shape=(
            jax.ShapeDtypeStruct((2, NP_, HD), jnp.float32),
            jax.ShapeDtypeStruct((2, NP_), jnp.float32),
        ),
    )(hin, w, asrc.reshape(1, D), adst.reshape(1, D))


def _sc_att_body(sd_hbm, srcf_hbm, dstf_hbm, zd_hbm, w_hbm, den_hbm,
                 s_v, d_v, srcf_v, dstf_v, wf_v, den_sh, dsem):
    cid = lax.axis_index("c")
    sid = lax.axis_index("s")

    pltpu.sync_copy(sd_hbm.at[0], s_v)
    pltpu.sync_copy(sd_hbm.at[1], d_v)
    pltpu.sync_copy(srcf_hbm.at[sid].at[cid], srcf_v)
    pltpu.sync_copy(dstf_hbm.at[sid].at[cid], dstf_v)
    pltpu.sync_copy(zd_hbm.at[pl.ds(sid * RPS, RPS)],
                    den_sh.at[pl.ds(sid * RPS, RPS)])
    plsc.subcore_barrier()

    @pl.loop(0, EH, step=16)
    def _att(j):
        si = srcf_v[pl.ds(j, 16)]
        di = dstf_v[pl.ds(j, 16)]
        sv = plsc.load_gather(s_v, [si])
        dv = plsc.load_gather(d_v, [di])
        e = sv + dv
        e = jnp.maximum(e, 0.2 * e)
        wf_v[pl.ds(j, 16)] = jnp.exp(e)

    pltpu.async_copy(wf_v, den_sh.at[dstf_v], dsem, add=True)
    pltpu.sync_copy(wf_v, w_hbm.at[sid].at[cid])
    pltpu.make_async_copy(wf_v, den_sh.at[dstf_v], dsem).wait()
    plsc.subcore_barrier()
    pltpu.sync_copy(den_sh.at[pl.ds(sid * RPS, RPS)],
                    den_hbm.at[cid].at[pl.ds(sid * RPS, RPS)])


def _sc_agg_body(h_hbm, src_hbm, dst_hbm, w4_hbm, zr_hbm, out_hbm,
                 src_v, dst_v, w_v, rows_v, h_sh, out_sh, gsem, ssem, wsem):
    cid = lax.axis_index("c")
    sid = lax.axis_index("s")

    pltpu.sync_copy(h_hbm.at[cid].at[pl.ds(sid * RPS, RPS)],
                    h_sh.at[pl.ds(sid * RPS, RPS)])
    pltpu.sync_copy(zr_hbm.at[pl.ds(sid * RPS, RPS)],
                    out_sh.at[pl.ds(sid * RPS, RPS)])
    plsc.subcore_barrier()

    def start_fetch(hh, kk, b):
        pltpu.async_copy(w4_hbm.at[sid].at[hh * CHK2 + kk], w_v.at[b],
                         wsem.at[b])
        pltpu.async_copy(h_sh.at[src_v.at[kk]], rows_v.at[b], gsem.at[b])

    def wait_fetch(b):
        pltpu.make_async_copy(w4_hbm.at[sid].at[0], w_v.at[b],
                              wsem.at[b]).wait()
        pltpu.make_async_copy(h_sh.at[src_v.at[0]], rows_v.at[b],
                              gsem.at[b]).wait()

    def wait_scatter(b):
        pltpu.make_async_copy(rows_v.at[b], out_sh.at[dst_v.at[0]],
                              ssem.at[b]).wait()

    for hh in range(2):
        pltpu.sync_copy(src_hbm.at[sid].at[pl.ds(hh * CHK2, CHK2)], src_v)
        pltpu.sync_copy(dst_hbm.at[sid].at[pl.ds(hh * CHK2, CHK2)], dst_v)

        for b in range(NB):
            start_fetch(hh, b, b)

        @pl.loop(0, NGRPH)
        def _grp(g):
            k0 = g * NB
            for b in range(NB):
                kk = k0 + b
                wait_fetch(b)
                @pl.loop(0, K, step=16)
                def _scale(gg):
                    w16 = w_v[b, pl.ds(gg, 16)]
                    for t in range(16):
                        a = w16[t]
                        for j in range(HD // 16):
                            sl = pl.ds(j * 16, 16)
                            rows_v[b, gg + t, sl] = rows_v[b, gg + t, sl] * a
                pltpu.async_copy(rows_v.at[b], out_sh.at[dst_v.at[kk]],
                                 ssem.at[b], add=True)
            for b in range(NB):
                kk2 = k0 + NB + b

                @pl.when(kk2 < CHK2)
                def _prefetch():
                    wait_scatter(b)
                    start_fetch(hh, kk2, b)

        for b in range(NB):
            wait_scatter(b)

    plsc.subcore_barrier()
    pltpu.sync_copy(out_sh.at[pl.ds(sid * RPS, RPS)],
                    out_hbm.at[cid].at[pl.ds(sid * RPS, RPS)])


def _sc_compiler_params():
    cp = pltpu.CompilerParams()
    fields = pltpu.CompilerParams.__dataclass_fields__
    if "needs_layout_passes" in fields:
        cp = dataclasses.replace(cp, needs_layout_passes=False)
    if "use_tc_tiling_on_sc" in fields:
        cp = dataclasses.replace(cp, use_tc_tiling_on_sc=False)
    return cp


def _sc_att(sd, srcf, dstf, zd):
    mesh = plsc.VectorSubcoreMesh(core_axis_name="c", subcore_axis_name="s")
    kfn = pl.kernel(
        _sc_att_body,
        out_type=(
            jax.ShapeDtypeStruct((NSUB, 2, EH), jnp.float32),
            jax.ShapeDtypeStruct((2, NP_), jnp.float32),
        ),
        mesh=mesh,
        scratch_types=[
            pltpu.VMEM((NP_,), jnp.float32),
            pltpu.VMEM((NP_,), jnp.float32),
            pltpu.VMEM((EH,), jnp.int32),
            pltpu.VMEM((EH,), jnp.int32),
            pltpu.VMEM((EH,), jnp.float32),
            pltpu.VMEM_SHARED((NP_,), jnp.float32),
            pltpu.SemaphoreType.DMA,
        ],
        compiler_params=_sc_compiler_params(),
    )
    return kfn(sd, srcf, dstf, zd)


def _sc_agg(h_parts, src3, dst3, w4, zr):
    mesh = plsc.VectorSubcoreMesh(core_axis_name="c", subcore_axis_name="s")
    kfn = pl.kernel(
        _sc_agg_body,
        out_type=jax.ShapeDtypeStruct((2, NP_, HD), jnp.float32),
        mesh=mesh,
        scratch_types=[
            pltpu.VMEM((CHK2, K), jnp.int32),
            pltpu.VMEM((CHK2, K), jnp.int32),
            pltpu.VMEM((NB, K), jnp.float32),
            pltpu.VMEM((NB, K, HD), jnp.float32),
            pltpu.VMEM_SHARED((NP_, HD), jnp.float32),
            pltpu.VMEM_SHARED((NP_, HD), jnp.float32),
            pltpu.SemaphoreType.DMA((NB,)),
            pltpu.SemaphoreType.DMA((NB,)),
            pltpu.SemaphoreType.DMA((NB,)),
        ],
        compiler_params=_sc_compiler_params(),
    )
    return kfn(h_parts, src3, dst3, w4, zr)


def _tc_norm_body(ep_ref, den_ref, b_ref, batchrow_ref, batch2d_ref,
                  h_ref, pmean_ref, pmax_ref):
    num = jnp.concatenate([ep_ref[0], ep_ref[1]], axis=1)
    den = den_ref[0] + den_ref[1]
    h = jnp.maximum(num / (den + 1e-16) + b_ref[...], 0.0)
    h_ref[...] = h
    segs = lax.broadcasted_iota(jnp.int32, (B, NP_), 0)
    mt = jnp.where(segs == batchrow_ref[...], 1.0, 0.0)
    s = jnp.dot(mt, h, preferred_element_type=jnp.float32)
    cnt = jnp.dot(mt, jnp.ones((NP_, 8), jnp.float32),
                  preferred_element_type=jnp.float32)[:, 0:1]
    pmean_ref[...] = s / jnp.maximum(cnt, 1.0)
    batch2d = batch2d_ref[...]
    for seg in range(B):
        m = batch2d == seg
        pmax_ref[seg, :] = jnp.max(jnp.where(m, h, 0.0), axis=0)


def _tc_post(ep, den2d, b, batch2d, batchrow):
    hr, pmean, pmax = pl.pallas_call(
        _tc_norm_body,
        out_shape=(
            jax.ShapeDtypeStruct((NP_, D), jnp.float32),
            jax.ShapeDtypeStruct((B, D), jnp.float32),
            jax.ShapeDtypeStruct((B, D), jnp.float32),
        ),
    )(ep, den2d, b.reshape(1, D), batchrow, batch2d)
    pool = jnp.concatenate([pmax, pmean], axis=1)
    return hr, pool


def _tc_head_body(p1_ref, p2_ref, p3_ref, l1w_ref, l1b_ref, l2w_ref, l2b_ref,
                  l3w_ref, l3b_ref, out_ref):
    g = p1_ref[...] + p2_ref[...] + p3_ref[...]
    g = jnp.maximum(jnp.dot(g, l1w_ref[...], preferred_element_type=jnp.float32)
                    + l1b_ref[...], 0.0)
    g = jnp.maximum(jnp.dot(g, l2w_ref[...], preferred_element_type=jnp.float32)
                    + l2b_ref[...], 0.0)
    lg = jnp.dot(g, l3w_ref[...], preferred_element_type=jnp.float32) + l3b_ref[...]
    cols = lax.broadcasted_iota(jnp.int32, (B, 128), 1)
    valid = cols < NC
    mx = jnp.max(jnp.where(valid, lg, -jnp.inf), axis=1, keepdims=True)
    ex = jnp.where(valid, jnp.exp(lg - mx), 0.0)
    lse = jnp.log(jnp.sum(ex, axis=1, keepdims=True))
    out_ref[...] = lg - mx - lse


def _tc_head(p1, p2, p3, l1w, l1b, l2w, l2b, l3wp, l3bp):
    return pl.pallas_call(
        _tc_head_body,
        out_shape=jax.ShapeDtypeStruct((B, 128), jnp.float32),
    )(p1, p2, p3, l1w, l1b.reshape(1, D), l2w, l2b.reshape(1, D // 2),
      l3wp, l3bp)


def kernel(x, pos, edge_index, batch, W1, asrc1, adst1, b1, W2, asrc2, adst2,
           b2, W3, asrc3, adst3, b3, L1w, L1b, L2w, L2b, L3w, L3b):
    loops = jnp.arange(N, dtype=jnp.int32)
    pad_e = jnp.full((EP - E - N,), N, jnp.int32)
    src3 = jnp.concatenate([edge_index[0], loops, pad_e]).reshape(NSUB, CHK, K)
    dst3 = jnp.concatenate([edge_index[1], loops, pad_e]).reshape(NSUB, CHK, K)
    srcf = src3.reshape(NSUB, 2, EH)
    dstf = dst3.reshape(NSUB, 2, EH)
    batch_p = jnp.concatenate([batch, jnp.full((NP_ - N,), B, jnp.int32)])
    batch2d = batch_p.reshape(NP_, 1)
    batchrow = batch_p.reshape(1, NP_)
    hin = jnp.pad(jnp.concatenate([x, pos], axis=1), ((0, NP_ - N), (0, 0)))
    zr = jnp.zeros((NP_, HD), jnp.float32)
    zd = jnp.zeros((NP_,), jnp.float32)
    l3wp = jnp.pad(L3w, ((0, 0), (0, 128 - NC)))
    l3bp = jnp.pad(L3b, (0, 128 - NC)).reshape(1, 128)

    h, sd = _tc_fwd(hin, W1, asrc1, adst1)
    w_e, den = _sc_att(sd, srcf, dstf, zd)
    ep = _sc_agg(h, src3, dst3, w_e.reshape(NSUB, CHK, K), zr)
    hr, p1 = _tc_post(ep, den.reshape(2, NP_, 1), b1, batch2d, batchrow)

    h, sd = _tc_fwd(hr, W2, asrc2, adst2)
    w_e, den = _sc_att(sd, srcf, dstf, zd)
    ep = _sc_agg(h, src3, dst3, w_e.reshape(NSUB, CHK, K), zr)
    hr, p2 = _tc_post(ep, den.reshape(2, NP_, 1), b2, batch2d, batchrow)

    h, sd = _tc_fwd(hr, W3, asrc3, adst3)
    w_e, den = _sc_att(sd, srcf, dstf, zd)
    ep = _sc_agg(h, src3, dst3, w_e.reshape(NSUB, CHK, K), zr)
    hr, p3 = _tc_post(ep, den.reshape(2, NP_, 1), b3, batch2d, batchrow)

    out = _tc_head(p1, p2, p3, L1w, L1b, L2w, L2b, l3wp, l3bp)
    return out[:, :NC]


# speedup vs baseline: 1.1297x; 1.1297x over previous
import dataclasses

import jax
import jax.numpy as jnp
from jax import lax
from jax.experimental import pallas as pl
from jax.experimental.pallas import tpu as pltpu
from jax.experimental.pallas import tpu_sc as plsc

N = 10000
E = 320000
B = 64
D = 128
NC = 10

NP_ = 10112
HD = 64
NSUB = 16
CHK = 164
CHK2 = CHK // 2
K = 128
EP = NSUB * CHK * K
EH = CHK2 * K
RPS = NP_ // 16
NB = 2
NGRPH = CHK2 // NB


def _tc_fwd_body(hin_ref, w_ref, asrc_ref, adst_ref, h_ref, sd_ref):
    h = jnp.dot(hin_ref[...], w_ref[...], preferred_element_type=jnp.float32)
    h_ref[0, :, :] = h[:, 0:HD]
    h_ref[1, :, :] = h[:, HD:D]
    sd_ref[0, :] = jnp.sum(h * asrc_ref[...], axis=1)
    sd_ref[1, :] = jnp.sum(h * adst_ref[...], axis=1)


def _tc_fwd(hin, w, asrc, adst):
    return pl.pallas_call(
        _tc_fwd_body,
        out_shape=(
            jax.ShapeDtypeStruct((2, NP_, HD), jnp.float32),
            jax.ShapeDtypeStruct((2, NP_), jnp.float32),
        ),
    )(hin, w, asrc.reshape(1, D), adst.reshape(1, D))


def _sc_att_body(sd_hbm, srcf_hbm, dstf_hbm, zd_hbm, w_hbm, den_hbm,
                 s_v, d_v, srcf_v, dstf_v, wf_v, den_sh, dsem):
    cid = lax.axis_index("c")
    sid = lax.axis_index("s")

    pltpu.sync_copy(sd_hbm.at[0], s_v)
    pltpu.sync_copy(sd_hbm.at[1], d_v)
    pltpu.sync_copy(srcf_hbm.at[sid].at[cid], srcf_v)
    pltpu.sync_copy(dstf_hbm.at[sid].at[cid], dstf_v)
    pltpu.sync_copy(zd_hbm.at[pl.ds(sid * RPS, RPS)],
                    den_sh.at[pl.ds(sid * RPS, RPS)])
    plsc.subcore_barrier()

    @pl.loop(0, EH, step=16)
    def _att(j):
        si = srcf_v[pl.ds(j, 16)]
        di = dstf_v[pl.ds(j, 16)]
        sv = plsc.load_gather(s_v, [si])
        dv = plsc.load_gather(d_v, [di])
        e = sv + dv
        e = jnp.maximum(e, 0.2 * e)
        wf_v[pl.ds(j, 16)] = jnp.exp(e)

    pltpu.async_copy(wf_v, den_sh.at[dstf_v], dsem, add=True)
    pltpu.sync_copy(wf_v, w_hbm.at[sid].at[cid])
    pltpu.make_async_copy(wf_v, den_sh.at[dstf_v], dsem).wait()
    plsc.subcore_barrier()
    pltpu.sync_copy(den_sh.at[pl.ds(sid * RPS, RPS)],
                    den_hbm.at[cid].at[pl.ds(sid * RPS, RPS)])


def _sc_agg_body(h_hbm, src_hbm, dst_hbm, w4_hbm, zr_hbm, out_hbm,
                 src_v, dst_v, w_v, rows_v, h_sh, out_sh, gsem, ssem, wsem):
    cid = lax.axis_index("c")
    sid = lax.axis_index("s")

    pltpu.sync_copy(h_hbm.at[cid].at[pl.ds(sid * RPS, RPS)],
                    h_sh.at[pl.ds(sid * RPS, RPS)])
    pltpu.sync_copy(zr_hbm.at[pl.ds(sid * RPS, RPS)],
                    out_sh.at[pl.ds(sid * RPS, RPS)])
    plsc.subcore_barrier()

    def start_fetch(hh, kk, b):
        pltpu.async_copy(w4_hbm.at[sid].at[hh * CHK2 + kk], w_v.at[b],
                         wsem.at[b])
        pltpu.async_copy(h_sh.at[src_v.at[kk]], rows_v.at[b], gsem.at[b])

    def wait_fetch(b):
        pltpu.make_async_copy(w4_hbm.at[sid].at[0], w_v.at[b],
                              wsem.at[b]).wait()
        pltpu.make_async_copy(h_sh.at[src_v.at[0]], rows_v.at[b],
                              gsem.at[b]).wait()

    def wait_scatter(b):
        pltpu.make_async_copy(rows_v.at[b], out_sh.at[dst_v.at[0]],
                              ssem.at[b]).wait()

    for hh in range(2):
        pltpu.sync_copy(src_hbm.at[sid].at[pl.ds(hh * CHK2, CHK2)], src_v)
        pltpu.sync_copy(dst_hbm.at[sid].at[pl.ds(hh * CHK2, CHK2)], dst_v)

        for b in range(NB):
            start_fetch(hh, b, b)

        @pl.loop(0, NGRPH)
        def _grp(g):
            k0 = g * NB
            for b in range(NB):
                kk = k0 + b
                wait_fetch(b)
                @pl.loop(0, K, step=16)
                def _scale(gg):
                    w16 = w_v[b, pl.ds(gg, 16)]
                    for t in range(16):
                        a = w16[t]
                        for j in range(HD // 16):
                            sl = pl.ds(j * 16, 16)
                            rows_v[b, gg + t, sl] = rows_v[b, gg + t, sl] * a
                pltpu.async_copy(rows_v.at[b], out_sh.at[dst_v.at[kk]],
                                 ssem.at[b], add=True)
            for b in range(NB):
                kk2 = k0 + NB + b

                @pl.when(kk2 < CHK2)
                def _prefetch():
                    wait_scatter(b)
                    start_fetch(hh, kk2, b)

        for b in range(NB):
            wait_scatter(b)

    plsc.subcore_barrier()
    pltpu.sync_copy(out_sh.at[pl.ds(sid * RPS, RPS)],
                    out_hbm.at[cid].at[pl.ds(sid * RPS, RPS)])


def _sc_compiler_params():
    cp = pltpu.CompilerParams()
    fields = pltpu.CompilerParams.__dataclass_fields__
    if "needs_layout_passes" in fields:
        cp = dataclasses.replace(cp, needs_layout_passes=False)
    if "use_tc_tiling_on_sc" in fields:
        cp = dataclasses.replace(cp, use_tc_tiling_on_sc=False)
    return cp


def _sc_att(sd, srcf, dstf, zd):
    mesh = plsc.VectorSubcoreMesh(core_axis_name="c", subcore_axis_name="s")
    kfn = pl.kernel(
        _sc_att_body,
        out_type=(
            jax.ShapeDtypeStruct((NSUB, 2, EH), jnp.float32),
            jax.ShapeDtypeStruct((2, NP_), jnp.float32),
        ),
        mesh=mesh,
        scratch_types=[
            pltpu.VMEM((NP_,), jnp.float32),
            pltpu.VMEM((NP_,), jnp.float32),
            pltpu.VMEM((EH,), jnp.int32),
            pltpu.VMEM((EH,), jnp.int32),
            pltpu.VMEM((EH,), jnp.float32),
            pltpu.VMEM_SHARED((NP_,), jnp.float32),
            pltpu.SemaphoreType.DMA,
        ],
        compiler_params=_sc_compiler_params(),
    )
    return kfn(sd, srcf, dstf, zd)


def _sc_agg(h_parts, src3, dst3, w4, zr):
    mesh = plsc.VectorSubcoreMesh(core_axis_name="c", subcore_axis_name="s")
    kfn = pl.kernel(
        _sc_agg_body,
        out_type=jax.ShapeDtypeStruct((2, NP_, HD), jnp.float32),
        mesh=mesh,
        scratch_types=[
            pltpu.VMEM((CHK2, K), jnp.int32),
            pltpu.VMEM((CHK2, K), jnp.int32),
            pltpu.VMEM((NB, K), jnp.float32),
            pltpu.VMEM((NB, K, HD), jnp.float32),
            pltpu.VMEM_SHARED((NP_, HD), jnp.float32),
            pltpu.VMEM_SHARED((NP_, HD), jnp.float32),
            pltpu.SemaphoreType.DMA((NB,)),
            pltpu.SemaphoreType.DMA((NB,)),
            pltpu.SemaphoreType.DMA((NB,)),
        ],
        compiler_params=_sc_compiler_params(),
    )
    return kfn(h_parts, src3, dst3, w4, zr)


def _tc_norm_body(ep_ref, den_ref, b_ref, h_ref):
    num = jnp.concatenate([ep_ref[0], ep_ref[1]], axis=1)
    den = den_ref[0] + den_ref[1]
    h_ref[...] = jnp.maximum(num / (den + 1e-16) + b_ref[...], 0.0)


def _tc_pool_body(h_ref, batch_ref, pmax_ref, pmean_ref):
    seg = pl.program_id(0)
    m = batch_ref[...] == seg
    hm = jnp.where(m, h_ref[...], 0.0)
    pmax_ref[0, 0, :] = jnp.max(hm, axis=0)
    cnt = jnp.sum(m.astype(jnp.float32))
    pmean_ref[0, 0, :] = jnp.sum(hm, axis=0) / jnp.maximum(cnt, 1.0)


def _tc_post(ep, den2d, b, batch2d, batchrow):
    hr = pl.pallas_call(
        _tc_norm_body,
        out_shape=jax.ShapeDtypeStruct((NP_, D), jnp.float32),
    )(ep, den2d, b.reshape(1, D))
    pmax, pmean = pl.pallas_call(
        _tc_pool_body,
        grid=(B,),
        in_specs=[
            pl.BlockSpec((NP_, D), lambda s: (0, 0)),
            pl.BlockSpec((NP_, 1), lambda s: (0, 0)),
        ],
        out_specs=(
            pl.BlockSpec((1, 1, D), lambda s: (s, 0, 0)),
            pl.BlockSpec((1, 1, D), lambda s: (s, 0, 0)),
        ),
        out_shape=(
            jax.ShapeDtypeStruct((B, 1, D), jnp.float32),
            jax.ShapeDtypeStruct((B, 1, D), jnp.float32),
        ),
    )(hr, batch2d)
    pool = jnp.concatenate([pmax.reshape(B, D), pmean.reshape(B, D)], axis=1)
    return hr, pool


def _tc_head_body(p1_ref, p2_ref, p3_ref, l1w_ref, l1b_ref, l2w_ref, l2b_ref,
                  l3w_ref, l3b_ref, out_ref):
    g = p1_ref[...] + p2_ref[...] + p3_ref[...]
    g = jnp.maximum(jnp.dot(g, l1w_ref[...], preferred_element_type=jnp.float32)
                    + l1b_ref[...], 0.0)
    g = jnp.maximum(jnp.dot(g, l2w_ref[...], preferred_element_type=jnp.float32)
                    + l2b_ref[...], 0.0)
    lg = jnp.dot(g, l3w_ref[...], preferred_element_type=jnp.float32) + l3b_ref[...]
    cols = lax.broadcasted_iota(jnp.int32, (B, 128), 1)
    valid = cols < NC
    mx = jnp.max(jnp.where(valid, lg, -jnp.inf), axis=1, keepdims=True)
    ex = jnp.where(valid, jnp.exp(lg - mx), 0.0)
    lse = jnp.log(jnp.sum(ex, axis=1, keepdims=True))
    out_ref[...] = lg - mx - lse


def _tc_head(p1, p2, p3, l1w, l1b, l2w, l2b, l3wp, l3bp):
    return pl.pallas_call(
        _tc_head_body,
        out_shape=jax.ShapeDtypeStruct((B, 128), jnp.float32),
    )(p1, p2, p3, l1w, l1b.reshape(1, D), l2w, l2b.reshape(1, D // 2),
      l3wp, l3bp)


def kernel(x, pos, edge_index, batch, W1, asrc1, adst1, b1, W2, asrc2, adst2,
           b2, W3, asrc3, adst3, b3, L1w, L1b, L2w, L2b, L3w, L3b):
    loops = jnp.arange(N, dtype=jnp.int32)
    pad_e = jnp.full((EP - E - N,), N, jnp.int32)
    src3 = jnp.concatenate([edge_index[0], loops, pad_e]).reshape(NSUB, CHK, K)
    dst3 = jnp.concatenate([edge_index[1], loops, pad_e]).reshape(NSUB, CHK, K)
    srcf = src3.reshape(NSUB, 2, EH)
    dstf = dst3.reshape(NSUB, 2, EH)
    batch_p = jnp.concatenate([batch, jnp.full((NP_ - N,), B, jnp.int32)])
    batch2d = batch_p.reshape(NP_, 1)
    batchrow = batch_p.reshape(1, NP_)
    hin = jnp.pad(jnp.concatenate([x, pos], axis=1), ((0, NP_ - N), (0, 0)))
    zr = jnp.zeros((NP_, HD), jnp.float32)
    zd = jnp.zeros((NP_,), jnp.float32)
    l3wp = jnp.pad(L3w, ((0, 0), (0, 128 - NC)))
    l3bp = jnp.pad(L3b, (0, 128 - NC)).reshape(1, 128)

    h, sd = _tc_fwd(hin, W1, asrc1, adst1)
    w_e, den = _sc_att(sd, srcf, dstf, zd)
    ep = _sc_agg(h, src3, dst3, w_e.reshape(NSUB, CHK, K), zr)
    hr, p1 = _tc_post(ep, den.reshape(2, NP_, 1), b1, batch2d, batchrow)

    h, sd = _tc_fwd(hr, W2, asrc2, adst2)
    w_e, den = _sc_att(sd, srcf, dstf, zd)
    ep = _sc_agg(h, src3, dst3, w_e.reshape(NSUB, CHK, K), zr)
    hr, p2 = _tc_post(ep, den.reshape(2, NP_, 1), b2, batch2d, batchrow)

    h, sd = _tc_fwd(hr, W3, asrc3, adst3)
    w_e, den = _sc_att(sd, srcf, dstf, zd)
    ep = _sc_agg(h, src3, dst3, w_e.reshape(NSUB, CHK, K), zr)
    hr, p3 = _tc_post(ep, den.reshape(2, NP_, 1), b3, batch2d, batchrow)

    out = _tc_head(p1, p2, p3, L1w, L1b, L2w, L2b, l3wp, l3bp)
    return out[:, :NC]


# speedup vs baseline: 1.1591x; 1.0261x over previous
import dataclasses

import jax
import jax.numpy as jnp
from jax import lax
from jax.experimental import pallas as pl
from jax.experimental.pallas import tpu as pltpu
from jax.experimental.pallas import tpu_sc as plsc

N = 10000
E = 320000
B = 64
D = 128
NC = 10

NP_ = 10112
HD = 64
NSUB = 16
CHK = 164
CHK2 = CHK // 2
K = 128
EP = NSUB * CHK * K
EH = CHK2 * K
RPS = NP_ // 16
NB = 2
NGRPH = CHK2 // NB


def _tc_fwd_body(hin_ref, w_ref, asrc_ref, adst_ref, h_ref, sd_ref):
    h = jnp.dot(hin_ref[...], w_ref[...], preferred_element_type=jnp.float32)
    h_ref[0, :, :] = h[:, 0:HD]
    h_ref[1, :, :] = h[:, HD:D]
    sd_ref[0, :] = jnp.sum(h * asrc_ref[...], axis=1)
    sd_ref[1, :] = jnp.sum(h * adst_ref[...], axis=1)


def _tc_fwd(hin, w, asrc, adst):
    return pl.pallas_call(
        _tc_fwd_body,
        out_shape=(
            jax.ShapeDtypeStruct((2, NP_, HD), jnp.float32),
            jax.ShapeDtypeStruct((2, NP_), jnp.float32),
        ),
    )(hin, w, asrc.reshape(1, D), adst.reshape(1, D))


def _sc_att_body(sd_hbm, srcf_hbm, dstf_hbm, zd_hbm, w_hbm, den_hbm,
                 s_v, d_v, srcf_v, dstf_v, wf_v, den_sh, dsem):
    cid = lax.axis_index("c")
    sid = lax.axis_index("s")

    pltpu.sync_copy(sd_hbm.at[0], s_v)
    pltpu.sync_copy(sd_hbm.at[1], d_v)
    pltpu.sync_copy(srcf_hbm.at[sid].at[cid], srcf_v)
    pltpu.sync_copy(dstf_hbm.at[sid].at[cid], dstf_v)
    pltpu.sync_copy(zd_hbm.at[pl.ds(sid * RPS, RPS)],
                    den_sh.at[pl.ds(sid * RPS, RPS)])
    plsc.subcore_barrier()

    @pl.loop(0, EH, step=16)
    def _att(j):
        si = srcf_v[pl.ds(j, 16)]
        di = dstf_v[pl.ds(j, 16)]
        sv = plsc.load_gather(s_v, [si])
        dv = plsc.load_gather(d_v, [di])
        e = sv + dv
        e = jnp.maximum(e, 0.2 * e)
        wf_v[pl.ds(j, 16)] = jnp.exp(e)

    pltpu.async_copy(wf_v, den_sh.at[dstf_v], dsem, add=True)
    pltpu.sync_copy(wf_v, w_hbm.at[sid].at[cid])
    pltpu.make_async_copy(wf_v, den_sh.at[dstf_v], dsem).wait()
    plsc.subcore_barrier()
    pltpu.sync_copy(den_sh.at[pl.ds(sid * RPS, RPS)],
                    den_hbm.at[cid].at[pl.ds(sid * RPS, RPS)])


def _sc_agg_body(h_hbm, src_hbm, dst_hbm, w4_hbm, zr_hbm, out_hbm,
                 src_v, dst_v, w_v, rows_v, h_sh, out_sh, gsem, ssem, wsem):
    cid = lax.axis_index("c")
    sid = lax.axis_index("s")

    pltpu.sync_copy(h_hbm.at[cid].at[pl.ds(sid * RPS, RPS)],
                    h_sh.at[pl.ds(sid * RPS, RPS)])
    pltpu.sync_copy(zr_hbm.at[pl.ds(sid * RPS, RPS)],
                    out_sh.at[pl.ds(sid * RPS, RPS)])
    plsc.subcore_barrier()

    def start_fetch(hh, kk, b):
        pltpu.async_copy(w4_hbm.at[sid].at[hh * CHK2 + kk], w_v.at[b],
                         wsem.at[b])
        pltpu.async_copy(h_sh.at[src_v.at[kk]], rows_v.at[b], gsem.at[b])

    def wait_fetch(b):
        pltpu.make_async_copy(w4_hbm.at[sid].at[0], w_v.at[b],
                              wsem.at[b]).wait()
        pltpu.make_async_copy(h_sh.at[src_v.at[0]], rows_v.at[b],
                              gsem.at[b]).wait()

    def wait_scatter(b):
        pltpu.make_async_copy(rows_v.at[b], out_sh.at[dst_v.at[0]],
                              ssem.at[b]).wait()

    for hh in range(2):
        pltpu.sync_copy(src_hbm.at[sid].at[pl.ds(hh * CHK2, CHK2)], src_v)
        pltpu.sync_copy(dst_hbm.at[sid].at[pl.ds(hh * CHK2, CHK2)], dst_v)

        for b in range(NB):
            start_fetch(hh, b, b)

        @pl.loop(0, NGRPH)
        def _grp(g):
            k0 = g * NB
            for b in range(NB):
                kk = k0 + b
                wait_fetch(b)
                @pl.loop(0, K, step=16)
                def _scale(gg):
                    w16 = w_v[b, pl.ds(gg, 16)]
                    for t in range(16):
                        a = w16[t]
                        for j in range(HD // 16):
                            sl = pl.ds(j * 16, 16)
                            rows_v[b, gg + t, sl] = rows_v[b, gg + t, sl] * a
                pltpu.async_copy(rows_v.at[b], out_sh.at[dst_v.at[kk]],
                                 ssem.at[b], add=True)
            for b in range(NB):
                kk2 = k0 + NB + b

                @pl.when(kk2 < CHK2)
                def _prefetch():
                    wait_scatter(b)
                    start_fetch(hh, kk2, b)

        for b in range(NB):
            wait_scatter(b)

    plsc.subcore_barrier()
    pltpu.sync_copy(out_sh.at[pl.ds(sid * RPS, RPS)],
                    out_hbm.at[cid].at[pl.ds(sid * RPS, RPS)])


def _sc_compiler_params():
    cp = pltpu.CompilerParams()
    fields = pltpu.CompilerParams.__dataclass_fields__
    if "needs_layout_passes" in fields:
        cp = dataclasses.replace(cp, needs_layout_passes=False)
    if "use_tc_tiling_on_sc" in fields:
        cp = dataclasses.replace(cp, use_tc_tiling_on_sc=False)
    return cp


def _sc_att(sd, srcf, dstf, zd):
    mesh = plsc.VectorSubcoreMesh(core_axis_name="c", subcore_axis_name="s")
    kfn = pl.kernel(
        _sc_att_body,
        out_type=(
            jax.ShapeDtypeStruct((NSUB, 2, EH), jnp.float32),
            jax.ShapeDtypeStruct((2, NP_), jnp.float32),
        ),
        mesh=mesh,
        scratch_types=[
            pltpu.VMEM((NP_,), jnp.float32),
            pltpu.VMEM((NP_,), jnp.float32),
            pltpu.VMEM((EH,), jnp.int32),
            pltpu.VMEM((EH,), jnp.int32),
            pltpu.VMEM((EH,), jnp.float32),
            pltpu.VMEM_SHARED((NP_,), jnp.float32),
            pltpu.SemaphoreType.DMA,
        ],
        compiler_params=_sc_compiler_params(),
    )
    return kfn(sd, srcf, dstf, zd)


def _sc_agg(h_parts, src3, dst3, w4, zr):
    mesh = plsc.VectorSubcoreMesh(core_axis_name="c", subcore_axis_name="s")
    kfn = pl.kernel(
        _sc_agg_body,
        out_type=jax.ShapeDtypeStruct((2, NP_, HD), jnp.float32),
        mesh=mesh,
        scratch_types=[
            pltpu.VMEM((CHK2, K), jnp.int32),
            pltpu.VMEM((CHK2, K), jnp.int32),
            pltpu.VMEM((NB, K), jnp.float32),
            pltpu.VMEM((NB, K, HD), jnp.float32),
            pltpu.VMEM_SHARED((NP_, HD), jnp.float32),
            pltpu.VMEM_SHARED((NP_, HD), jnp.float32),
            pltpu.SemaphoreType.DMA((NB,)),
            pltpu.SemaphoreType.DMA((NB,)),
            pltpu.SemaphoreType.DMA((NB,)),
        ],
        compiler_params=_sc_compiler_params(),
    )
    return kfn(h_parts, src3, dst3, w4, zr)


def _tc_norm_body(ep_ref, den_ref, b_ref, batchrow_ref, h_ref, pmean_ref):
    num = jnp.concatenate([ep_ref[0], ep_ref[1]], axis=1)
    den = den_ref[0] + den_ref[1]
    h = jnp.maximum(num / (den + 1e-16) + b_ref[...], 0.0)
    h_ref[...] = h
    segs = lax.broadcasted_iota(jnp.int32, (B, NP_), 0)
    mt = jnp.where(segs == batchrow_ref[...], 1.0, 0.0)
    s = jnp.dot(mt, h, preferred_element_type=jnp.float32)
    cnt = jnp.dot(mt, jnp.ones((NP_, 8), jnp.float32),
                  preferred_element_type=jnp.float32)[:, 0:1]
    pmean_ref[...] = s / jnp.maximum(cnt, 1.0)


def _tc_maxpool_body(h_ref, batch_ref, pmax_ref):
    seg = pl.program_id(0)
    m = batch_ref[...] == seg
    pmax_ref[0, 0, :] = jnp.max(jnp.where(m, h_ref[...], 0.0), axis=0)


def _tc_post(ep, den2d, b, batch2d, batchrow):
    hr, pmean = pl.pallas_call(
        _tc_norm_body,
        out_shape=(
            jax.ShapeDtypeStruct((NP_, D), jnp.float32),
            jax.ShapeDtypeStruct((B, D), jnp.float32),
        ),
    )(ep, den2d, b.reshape(1, D), batchrow)
    pmax = pl.pallas_call(
        _tc_maxpool_body,
        grid=(B,),
        in_specs=[
            pl.BlockSpec((NP_, D), lambda s: (0, 0)),
            pl.BlockSpec((NP_, 1), lambda s: (0, 0)),
        ],
        out_specs=pl.BlockSpec((1, 1, D), lambda s: (s, 0, 0)),
        out_shape=jax.ShapeDtypeStruct((B, 1, D), jnp.float32),
    )(hr, batch2d)
    pool = jnp.concatenate([pmax.reshape(B, D), pmean], axis=1)
    return hr, pool


def _tc_head_body(p1_ref, p2_ref, p3_ref, l1w_ref, l1b_ref, l2w_ref, l2b_ref,
                  l3w_ref, l3b_ref, out_ref):
    g = p1_ref[...] + p2_ref[...] + p3_ref[...]
    g = jnp.maximum(jnp.dot(g, l1w_ref[...], preferred_element_type=jnp.float32)
                    + l1b_ref[...], 0.0)
    g = jnp.maximum(jnp.dot(g, l2w_ref[...], preferred_element_type=jnp.float32)
                    + l2b_ref[...], 0.0)
    lg = jnp.dot(g, l3w_ref[...], preferred_element_type=jnp.float32) + l3b_ref[...]
    cols = lax.broadcasted_iota(jnp.int32, (B, 128), 1)
    valid = cols < NC
    mx = jnp.max(jnp.where(valid, lg, -jnp.inf), axis=1, keepdims=True)
    ex = jnp.where(valid, jnp.exp(lg - mx), 0.0)
    lse = jnp.log(jnp.sum(ex, axis=1, keepdims=True))
    out_ref[...] = lg - mx - lse


def _tc_head(p1, p2, p3, l1w, l1b, l2w, l2b, l3wp, l3bp):
    return pl.pallas_call(
        _tc_head_body,
        out_shape=jax.ShapeDtypeStruct((B, 128), jnp.float32),
    )(p1, p2, p3, l1w, l1b.reshape(1, D), l2w, l2b.reshape(1, D // 2),
      l3wp, l3bp)


def kernel(x, pos, edge_index, batch, W1, asrc1, adst1, b1, W2, asrc2, adst2,
           b2, W3, asrc3, adst3, b3, L1w, L1b, L2w, L2b, L3w, L3b):
    loops = jnp.arange(N, dtype=jnp.int32)
    pad_e = jnp.full((EP - E - N,), N, jnp.int32)
    src3 = jnp.concatenate([edge_index[0], loops, pad_e]).reshape(NSUB, CHK, K)
    dst3 = jnp.concatenate([edge_index[1], loops, pad_e]).reshape(NSUB, CHK, K)
    srcf = src3.reshape(NSUB, 2, EH)
    dstf = dst3.reshape(NSUB, 2, EH)
    batch_p = jnp.concatenate([batch, jnp.full((NP_ - N,), B, jnp.int32)])
    batch2d = batch_p.reshape(NP_, 1)
    batchrow = batch_p.reshape(1, NP_)
    hin = jnp.pad(jnp.concatenate([x, pos], axis=1), ((0, NP_ - N), (0, 0)))
    zr = jnp.zeros((NP_, HD), jnp.float32)
    zd = jnp.zeros((NP_,), jnp.float32)
    l3wp = jnp.pad(L3w, ((0, 0), (0, 128 - NC)))
    l3bp = jnp.pad(L3b, (0, 128 - NC)).reshape(1, 128)

    h, sd = _tc_fwd(hin, W1, asrc1, adst1)
    w_e, den = _sc_att(sd, srcf, dstf, zd)
    ep = _sc_agg(h, src3, dst3, w_e.reshape(NSUB, CHK, K), zr)
    hr, p1 = _tc_post(ep, den.reshape(2, NP_, 1), b1, batch2d, batchrow)

    h, sd = _tc_fwd(hr, W2, asrc2, adst2)
    w_e, den = _sc_att(sd, srcf, dstf, zd)
    ep = _sc_agg(h, src3, dst3, w_e.reshape(NSUB, CHK, K), zr)
    hr, p2 = _tc_post(ep, den.reshape(2, NP_, 1), b2, batch2d, batchrow)

    h, sd = _tc_fwd(hr, W3, asrc3, adst3)
    w_e, den = _sc_att(sd, srcf, dstf, zd)
    ep = _sc_agg(h, src3, dst3, w_e.reshape(NSUB, CHK, K), zr)
    hr, p3 = _tc_post(ep, den.reshape(2, NP_, 1), b3, batch2d, batchrow)

    out = _tc_head(p1, p2, p3, L1w, L1b, L2w, L2b, l3wp, l3bp)
    return out[:, :NC]
